# baseline (speedup 1.0000x reference)
import functools

import jax
import jax.numpy as jnp
from jax import lax
from jax.experimental import pallas as pl
from jax.experimental.pallas import tpu as pltpu
from jax.experimental.pallas import tpu_sc as plsc

D_MODEL = 64
SCALE = 8.0
CHUNK = 128
NUM_WORKERS = 32


def kernel(x, table):
    B = x.shape[0] * x.shape[1]
    idx = x.reshape(B // CHUNK, CHUNK).astype(jnp.int32)
    rows_per_w = B // NUM_WORKERS
    chunks_per_w = rows_per_w // CHUNK

    mesh = plsc.VectorSubcoreMesh(core_axis_name="c", subcore_axis_name="s")

    @functools.partial(
        pl.kernel,
        mesh=mesh,
        compiler_params=pltpu.CompilerParams(use_tc_tiling_on_sc=False),
        out_type=jax.ShapeDtypeStruct((B, D_MODEL), jnp.float32),
        scratch_types=[
            pltpu.VMEM((chunks_per_w, CHUNK), jnp.int32),
            pltpu.VMEM((CHUNK, D_MODEL), jnp.float32),
            pltpu.SemaphoreType.DMA,
        ],
    )
    def emb_kernel(idx_hbm, table_hbm, out_hbm, idx_v, rows_v, sem):
        wid = lax.axis_index("s") * 2 + lax.axis_index("c")
        cbase = wid * chunks_per_w
        pltpu.sync_copy(idx_hbm.at[pl.ds(cbase, chunks_per_w)], idx_v)

        def chunk_body(g, carry):
            pltpu.async_copy(table_hbm.at[idx_v.at[g]], rows_v, sem).wait()

            def scale_body(i, c):
                for j in range(D_MODEL // 16):
                    sl = pl.ds(j * 16, 16)
                    rows_v[i, sl] = rows_v[i, sl] * SCALE
                return c

            lax.fori_loop(0, CHUNK, scale_body, 0)
            row0 = (cbase + g) * CHUNK
            pltpu.sync_copy(rows_v, out_hbm.at[pl.ds(row0, CHUNK)])
            return carry

        lax.fori_loop(0, chunks_per_w, chunk_body, 0)

    out = emb_kernel(idx, table)
    return out.reshape(x.shape[0], x.shape[1], D_MODEL)


# speedup vs baseline: 1.2094x; 1.2094x over previous
import functools

import jax
import jax.numpy as jnp
from jax import lax
from jax.experimental import pallas as pl
from jax.experimental.pallas import tpu as pltpu
from jax.experimental.pallas import tpu_sc as plsc

D_MODEL = 64
SCALE = 8.0
CHUNK = 128
NBUF = 4
NUM_WORKERS = 32


def kernel(x, table):
    B = x.shape[0] * x.shape[1]
    idx = x.reshape(B // CHUNK, CHUNK).astype(jnp.int32)
    rows_per_w = B // NUM_WORKERS
    chunks_per_w = rows_per_w // CHUNK

    mesh = plsc.VectorSubcoreMesh(core_axis_name="c", subcore_axis_name="s")

    @functools.partial(
        pl.kernel,
        mesh=mesh,
        compiler_params=pltpu.CompilerParams(use_tc_tiling_on_sc=False),
        out_type=jax.ShapeDtypeStruct((B, D_MODEL), jnp.float32),
        scratch_types=[
            pltpu.VMEM((chunks_per_w, CHUNK), jnp.int32),
            pltpu.VMEM((NBUF, CHUNK, D_MODEL), jnp.float32),
            [pltpu.SemaphoreType.DMA] * NBUF,
            [pltpu.SemaphoreType.DMA] * NBUF,
        ],
    )
    def emb_kernel(idx_hbm, table_hbm, out_hbm, idx_v, rows_v, gsem, ssem):
        wid = lax.axis_index("s") * 2 + lax.axis_index("c")
        cbase = wid * chunks_per_w
        pltpu.sync_copy(idx_hbm.at[pl.ds(cbase, chunks_per_w)], idx_v)

        for k in range(NBUF - 1):
            pltpu.make_async_copy(
                table_hbm.at[idx_v.at[k]], rows_v.at[k], gsem[k]
            ).start()

        def outer(it, carry):
            for k in range(NBUF):
                g = it * NBUF + k
                buf = rows_v.at[k]
                pltpu.make_async_copy(
                    table_hbm.at[idx_v.at[0]], buf, gsem[k]
                ).wait()

                @plsc.parallel_loop(0, CHUNK, unroll=8)
                def scale_row(i):
                    for j in range(D_MODEL // 16):
                        sl = pl.ds(j * 16, 16)
                        buf[i, sl] = buf[i, sl] * SCALE

                row0 = (cbase + g) * CHUNK
                pltpu.make_async_copy(
                    buf, out_hbm.at[pl.ds(row0, CHUNK)], ssem[k]
                ).start()

                kn = (k + NBUF - 1) % NBUF
                gn = g + NBUF - 1

                @pl.when(gn < chunks_per_w)
                def _refill():
                    @pl.when(g >= 1)
                    def _drain_store():
                        pltpu.make_async_copy(
                            rows_v.at[kn],
                            out_hbm.at[pl.ds(0, CHUNK)],
                            ssem[kn],
                        ).wait()

                    pltpu.make_async_copy(
                        table_hbm.at[idx_v.at[gn]], rows_v.at[kn], gsem[kn]
                    ).start()

            return carry

        lax.fori_loop(0, chunks_per_w // NBUF, outer, 0)

        for k in range(NBUF):
            pltpu.make_async_copy(
                rows_v.at[k], out_hbm.at[pl.ds(0, CHUNK)], ssem[k]
            ).wait()

    out = emb_kernel(idx, table)
    return out.reshape(x.shape[0], x.shape[1], D_MODEL)
